# baseline (speedup 1.0000x reference)
import jax
import jax.numpy as jnp
from jax import lax
from jax.experimental import pallas as pl
from jax.experimental.pallas import tpu as pltpu
from jax.experimental.pallas import tpu_sc as plsc

N_NODES_C = 10000
E_INC = 320000
D_C = 128

NC = 2
NS = 16
NW = NC * NS
LANES = 16
DG = D_C // LANES

CHUNK = E_INC // NW
BATCH = 400
SUB = 80
NSUB = BATCH // SUB
NBATCH = CHUNK // BATCH
NEG_INF = float("-inf")


RING = 8


def _sc_body(table_hbm, he_hbm, seg_hbm, out_hbm, srows_hbm, ssegs_hbm,
             seg_v, he_v, rows_v, rowring, rowbuf, segbuf, accbuf, state,
             gsem, fsem, isem):
    wid = lax.axis_index("c") * NS + lax.axis_index("s")
    start = wid * CHUNK

    cp_seg = pltpu.make_async_copy(seg_hbm.at[pl.ds(start, CHUNK)],
                                   seg_v.at[pl.ds(0, CHUNK)], isem)
    cp_he = pltpu.make_async_copy(he_hbm.at[pl.ds(start, CHUNK)], he_v, isem)
    cp_seg.start()
    cp_he.start()
    cp_seg.wait()
    cp_he.wait()

    first_seg = seg_v[pl.ds(0, LANES)][0]
    neg = jnp.full((LANES,), NEG_INF, dtype=jnp.float32)

    def emit_side(cur, accs, side_slot):
        for c in range(DG):
            rowbuf[pl.ds(c * LANES, LANES)] = accs[c]
        segbuf[...] = jnp.full((LANES,), cur, dtype=jnp.int32)
        pltpu.sync_copy(rowbuf, srows_hbm.at[2 * wid + side_slot])
        pltpu.sync_copy(segbuf, ssegs_hbm.at[2 * wid + side_slot])

    def flush_direct(cur, nflush, accs):
        slot = lax.rem(nflush, RING)

        @pl.when(nflush >= RING)
        def _():
            pltpu.make_async_copy(rowring.at[pl.ds(slot * D_C, D_C)],
                                  out_hbm.at[cur], fsem.at[slot]).wait()

        for c in range(DG):
            rowring[pl.ds(slot * D_C + c * LANES, LANES)] = accs[c]
        pltpu.make_async_copy(rowring.at[pl.ds(slot * D_C, D_C)],
                              out_hbm.at[cur], fsem.at[slot]).start()

    def fire(k, buf):
        for t in range(NSUB):
            pltpu.make_async_copy(
                table_hbm.at[he_v.at[pl.ds(k * BATCH + t * SUB, SUB)]],
                rows_v.at[pl.ds(buf * BATCH + t * SUB, SUB)],
                gsem.at[buf]).start()

    def drain(buf):
        for t in range(NSUB):
            pltpu.make_async_copy(
                table_hbm.at[he_v.at[pl.ds(t * SUB, SUB)]],
                rows_v.at[pl.ds(buf * BATCH + t * SUB, SUB)],
                gsem.at[buf]).wait()

    fire(0, 0)

    def batch_body(k, carry):
        buf = lax.rem(k, 2)
        drain(buf)

        @pl.when(k + 1 < NBATCH)
        def _():
            fire(k + 1, 1 - buf)

        def entry(jj, i0, ecarry):
            j = i0 + jj
            cur, nflush = ecarry[0], ecarry[1]
            accs = list(ecarry[2:])
            seg_j = seg_v[pl.ds(j, LANES)][0]
            new_seg = seg_j != cur
            direct = jnp.logical_and(new_seg, cur != first_seg)

            @pl.when(jnp.logical_and(new_seg, cur == first_seg))
            def _():
                emit_side(cur, accs, 0)

            @pl.when(direct)
            def _():
                flush_direct(cur, nflush, accs)

            nflush = nflush + direct.astype(jnp.int32)
            new_accs = []
            for c in range(DG):
                row_c = rows_v[buf * BATCH + j - k * BATCH,
                               pl.ds(c * LANES, LANES)]
                base = jnp.where(new_seg, neg, accs[c])
                new_accs.append(jnp.maximum(base, row_c))
            cur = jnp.where(new_seg, seg_j, cur)
            return (cur, nflush, *new_accs)

        def group(g, _):
            i = k * BATCH + g * LANES
            sv = seg_v[pl.ds(i, LANES)]
            lo = seg_v[pl.ds(jnp.maximum(i - 1, 0), LANES)][0]
            hi = sv[LANES - 1]
            uniform = jnp.logical_and(lo == hi, i > 0)

            @pl.when(uniform)
            def _():
                row0 = buf * BATCH + g * LANES
                for c in range(DG):
                    r = [rows_v[row0 + t, pl.ds(c * LANES, LANES)]
                         for t in range(LANES)]
                    while len(r) > 1:
                        r = [jnp.maximum(r[2 * q], r[2 * q + 1])
                             for q in range(len(r) // 2)]
                    accbuf[pl.ds(c * LANES, LANES)] = jnp.maximum(
                        accbuf[pl.ds(c * LANES, LANES)], r[0])

            svt = [sv[t] for t in range(LANES)]
            all_two = jnp.logical_or(svt[0] == lo, svt[0] == hi)
            for t in range(1, LANES):
                all_two = jnp.logical_and(
                    all_two, jnp.logical_or(svt[t] == lo, svt[t] == hi))
            one_b = jnp.logical_and(
                jnp.logical_and(jnp.logical_not(uniform), all_two), i > 0)

            @pl.when(one_b)
            def _():
                p = svt[0].astype(jnp.int32) * 0
                for t in range(LANES):
                    p = p + (svt[t] == lo).astype(jnp.int32)
                cur0, nflush0 = state[0], state[1]
                row0 = buf * BATCH + g * LANES
                tlt = [t < p for t in range(LANES)]
                acc_a = []
                acc_b = []
                for c in range(DG):
                    r = [rows_v[row0 + t, pl.ds(c * LANES, LANES)]
                         for t in range(LANES)]
                    ra = [jnp.where(tlt[t], r[t], neg)
                          for t in range(LANES)]
                    rb = [jnp.where(tlt[t], neg, r[t])
                          for t in range(LANES)]
                    while len(ra) > 1:
                        ra = [jnp.maximum(ra[2 * q], ra[2 * q + 1])
                              for q in range(len(ra) // 2)]
                        rb = [jnp.maximum(rb[2 * q], rb[2 * q + 1])
                              for q in range(len(rb) // 2)]
                    acc_a.append(jnp.maximum(
                        accbuf[pl.ds(c * LANES, LANES)], ra[0]))
                    acc_b.append(rb[0])

                @pl.when(cur0 == first_seg)
                def _():
                    emit_side(cur0, acc_a, 0)

                direct = cur0 != first_seg

                @pl.when(direct)
                def _():
                    flush_direct(cur0, nflush0, acc_a)

                state[0] = hi
                state[1] = nflush0 + direct.astype(jnp.int32)
                for c in range(DG):
                    accbuf[pl.ds(c * LANES, LANES)] = acc_b[c]

            serial_pred = jnp.logical_not(jnp.logical_or(uniform, one_b))

            @pl.when(serial_pred)
            def _():
                cur0, nflush0 = state[0], state[1]
                accs0 = [accbuf[pl.ds(c * LANES, LANES)]
                         for c in range(DG)]
                cur1, nflush1, *accs1 = lax.fori_loop(
                    0, LANES, lambda jj, ec: entry(jj, i, ec),
                    (cur0, nflush0, *accs0))
                state[0] = cur1
                state[1] = nflush1
                for c in range(DG):
                    accbuf[pl.ds(c * LANES, LANES)] = accs1[c]

            return 0

        return lax.fori_loop(0, BATCH // LANES, group, carry)

    state[0] = first_seg
    state[1] = jnp.int32(0)
    for c in range(DG):
        accbuf[pl.ds(c * LANES, LANES)] = neg
    lax.fori_loop(0, NBATCH, batch_body, 0)
    cur = state[0]
    nflush = state[1]
    accs = [accbuf[pl.ds(c * LANES, LANES)] for c in range(DG)]

    @pl.when(cur == first_seg)
    def _():
        emit_side(cur, accs, 0)

    emit_side(cur, accs, 1)

    for s in range(RING):
        @pl.when(s < nflush)
        def _():
            pltpu.make_async_copy(rowring.at[pl.ds(s * D_C, D_C)],
                                  out_hbm.at[0], fsem.at[s]).wait()


def _tc_fixup_body(ssegs_smem, draft_any, srows_vmem, ssegs_vmem, out_any,
                   merged_vmem, sem):
    del draft_any
    segs = ssegs_vmem[:, 0:1]

    def merge(j, acc):
        seg_col = ssegs_vmem[pl.ds(j, 1), 0:1]
        mask = segs == seg_col
        row_j = srows_vmem[pl.ds(j, 1), :]
        return jnp.maximum(acc, jnp.where(mask, row_j, NEG_INF))

    merged_vmem[...] = lax.fori_loop(
        0, 2 * NW, merge,
        jnp.full((2 * NW, D_C), NEG_INF, dtype=jnp.float32))

    def scatter(i, _):
        seg_i = ssegs_smem[i, 0]
        pltpu.make_async_copy(merged_vmem.at[i], out_any.at[seg_i],
                              sem.at[i]).start()
        return 0

    lax.fori_loop(0, 2 * NW, scatter, 0)

    def drain(i, _):
        seg_i = ssegs_smem[i, 0]
        pltpu.make_async_copy(merged_vmem.at[i], out_any.at[seg_i],
                              sem.at[i]).wait()
        return 0

    lax.fori_loop(0, 2 * NW, drain, 0)


@jax.jit
def _run(table, hyperedge_ids, segment_ids):
    mesh = plsc.VectorSubcoreMesh(core_axis_name="c", subcore_axis_name="s",
                                  num_cores=NC, num_subcores=NS)
    sc = pl.kernel(
        _sc_body,
        out_type=(
            jax.ShapeDtypeStruct((N_NODES_C, D_C), jnp.float32),
            jax.ShapeDtypeStruct((2 * NW, D_C), jnp.float32),
            jax.ShapeDtypeStruct((2 * NW, LANES), jnp.int32),
        ),
        mesh=mesh,
        scratch_types=[
            pltpu.VMEM((CHUNK + LANES,), jnp.int32),
            pltpu.VMEM((CHUNK,), jnp.int32),
            pltpu.VMEM((2 * BATCH, D_C), jnp.float32),
            pltpu.VMEM((RING * D_C,), jnp.float32),
            pltpu.VMEM((D_C,), jnp.float32),
            pltpu.VMEM((LANES,), jnp.int32),
            pltpu.VMEM((D_C,), jnp.float32),
            pltpu.SMEM((2,), jnp.int32),
            pltpu.SemaphoreType.DMA((2,)),
            pltpu.SemaphoreType.DMA((RING,)),
            pltpu.SemaphoreType.DMA,
        ],
    )
    draft, srows, ssegs = sc(table, hyperedge_ids, segment_ids)

    fix = pl.pallas_call(
        _tc_fixup_body,
        out_shape=jax.ShapeDtypeStruct((N_NODES_C, D_C), jnp.float32),
        in_specs=[
            pl.BlockSpec(memory_space=pltpu.SMEM),
            pl.BlockSpec(memory_space=pl.ANY),
            pl.BlockSpec(memory_space=pltpu.VMEM),
            pl.BlockSpec(memory_space=pltpu.VMEM),
        ],
        out_specs=pl.BlockSpec(memory_space=pl.ANY),
        scratch_shapes=[
            pltpu.VMEM((2 * NW, D_C), jnp.float32),
            pltpu.SemaphoreType.DMA((2 * NW,)),
        ],
        input_output_aliases={1: 0},
    )
    return fix(ssegs, draft, srows, ssegs)


def kernel(table, hyperedge_ids, segment_ids):
    return _run(table, hyperedge_ids, segment_ids)


# speedup vs baseline: 1.1789x; 1.1789x over previous
import jax
import jax.numpy as jnp
from jax import lax
from jax.experimental import pallas as pl
from jax.experimental.pallas import tpu as pltpu
from jax.experimental.pallas import tpu_sc as plsc

N_NODES_C = 10000
E_INC = 320000
D_C = 128

NC = 2
NS = 16
NW = NC * NS
LANES = 16
DG = D_C // LANES

CHUNK = E_INC // NW
BATCH = 400
SUB = 80
NSUB = BATCH // SUB
NBATCH = CHUNK // BATCH
NEG_INF = float("-inf")


RING = 8


def _sc_body(table_hbm, he_hbm, seg_hbm, out_hbm, srows_hbm, ssegs_hbm,
             seg_v, he_v, rows_v, rowring, rowbuf, segbuf, accbuf, state,
             gsem, fsem, isem):
    wid = lax.axis_index("c") * NS + lax.axis_index("s")
    start = wid * CHUNK

    cp_seg = pltpu.make_async_copy(seg_hbm.at[pl.ds(start, CHUNK)],
                                   seg_v.at[pl.ds(0, CHUNK)], isem)
    cp_he = pltpu.make_async_copy(he_hbm.at[pl.ds(start, CHUNK)], he_v, isem)
    cp_seg.start()
    cp_he.start()
    cp_seg.wait()
    cp_he.wait()

    first_seg = seg_v[pl.ds(0, LANES)][0]
    neg = jnp.full((LANES,), NEG_INF, dtype=jnp.float32)

    def emit_side(cur, accs, side_slot):
        for c in range(DG):
            rowbuf[pl.ds(c * LANES, LANES)] = accs[c]
        segbuf[...] = jnp.full((LANES,), cur, dtype=jnp.int32)
        pltpu.sync_copy(rowbuf, srows_hbm.at[2 * wid + side_slot])
        pltpu.sync_copy(segbuf, ssegs_hbm.at[2 * wid + side_slot])

    def flush_direct(cur, nflush, accs):
        slot = lax.rem(nflush, RING)

        @pl.when(nflush >= RING)
        def _():
            pltpu.make_async_copy(rowring.at[pl.ds(slot * D_C, D_C)],
                                  out_hbm.at[cur], fsem.at[slot]).wait()

        for c in range(DG):
            rowring[pl.ds(slot * D_C + c * LANES, LANES)] = accs[c]
        pltpu.make_async_copy(rowring.at[pl.ds(slot * D_C, D_C)],
                              out_hbm.at[cur], fsem.at[slot]).start()

    def fire(k, buf):
        for t in range(NSUB):
            pltpu.make_async_copy(
                table_hbm.at[he_v.at[pl.ds(k * BATCH + t * SUB, SUB)]],
                rows_v.at[pl.ds(buf * BATCH + t * SUB, SUB)],
                gsem.at[buf]).start()

    def drain(buf):
        for t in range(NSUB):
            pltpu.make_async_copy(
                table_hbm.at[he_v.at[pl.ds(t * SUB, SUB)]],
                rows_v.at[pl.ds(buf * BATCH + t * SUB, SUB)],
                gsem.at[buf]).wait()

    fire(0, 0)

    def batch_body(k, carry):
        buf = lax.rem(k, 2)
        drain(buf)

        @pl.when(k + 1 < NBATCH)
        def _():
            fire(k + 1, 1 - buf)

        def entry(jj, i0, ecarry):
            j = i0 + jj
            cur, nflush = ecarry[0], ecarry[1]
            accs = list(ecarry[2:])
            seg_j = seg_v[pl.ds(j, LANES)][0]
            new_seg = seg_j != cur
            direct = jnp.logical_and(new_seg, cur != first_seg)

            @pl.when(jnp.logical_and(new_seg, cur == first_seg))
            def _():
                emit_side(cur, accs, 0)

            @pl.when(direct)
            def _():
                flush_direct(cur, nflush, accs)

            nflush = nflush + direct.astype(jnp.int32)
            new_accs = []
            for c in range(DG):
                row_c = rows_v[buf * BATCH + j - k * BATCH,
                               pl.ds(c * LANES, LANES)]
                base = jnp.where(new_seg, neg, accs[c])
                new_accs.append(jnp.maximum(base, row_c))
            cur = jnp.where(new_seg, seg_j, cur)
            return (cur, nflush, *new_accs)

        def group(g, _):
            i = k * BATCH + g * LANES
            sv = seg_v[pl.ds(i, LANES)]
            psv = seg_v[pl.ds(jnp.maximum(i - 1, 0), LANES)]
            lo = psv[0]
            hi = sv[LANES - 1]
            uniform = jnp.logical_and(lo == hi, i > 0)
            svt = [sv[t] for t in range(LANES)]
            all_two = jnp.logical_or(svt[0] == lo, svt[0] == hi)
            for t in range(1, LANES):
                all_two = jnp.logical_and(
                    all_two, jnp.logical_or(svt[t] == lo, svt[t] == hi))
            one_b = jnp.logical_and(
                jnp.logical_and(jnp.logical_not(uniform), all_two), i > 0)

            @pl.when(uniform)
            def _():
                row0 = buf * BATCH + g * LANES
                for c in range(DG):
                    r = [rows_v[row0 + t, pl.ds(c * LANES, LANES)]
                         for t in range(LANES)]
                    while len(r) > 1:
                        r = [jnp.maximum(r[2 * q], r[2 * q + 1])
                             for q in range(len(r) // 2)]
                    accbuf[pl.ds(c * LANES, LANES)] = jnp.maximum(
                        accbuf[pl.ds(c * LANES, LANES)], r[0])

            @pl.when(one_b)
            def _():
                p = svt[0].astype(jnp.int32) * 0
                for t in range(LANES):
                    p = p + (svt[t] == lo).astype(jnp.int32)
                cur0, nflush0 = state[0], state[1]
                row0 = buf * BATCH + g * LANES
                tlt = [t < p for t in range(LANES)]
                acc_a = []
                acc_b = []
                for c in range(DG):
                    r = [rows_v[row0 + t, pl.ds(c * LANES, LANES)]
                         for t in range(LANES)]
                    ra = [jnp.where(tlt[t], r[t], neg) for t in range(LANES)]
                    rb = [jnp.where(tlt[t], neg, r[t]) for t in range(LANES)]
                    while len(ra) > 1:
                        ra = [jnp.maximum(ra[2 * q], ra[2 * q + 1])
                              for q in range(len(ra) // 2)]
                        rb = [jnp.maximum(rb[2 * q], rb[2 * q + 1])
                              for q in range(len(rb) // 2)]
                    acc_a.append(jnp.maximum(
                        accbuf[pl.ds(c * LANES, LANES)], ra[0]))
                    acc_b.append(rb[0])

                @pl.when(cur0 == first_seg)
                def _():
                    emit_side(cur0, acc_a, 0)

                direct = cur0 != first_seg

                @pl.when(direct)
                def _():
                    flush_direct(cur0, nflush0, acc_a)

                state[0] = hi
                state[1] = nflush0 + direct.astype(jnp.int32)
                for c in range(DG):
                    accbuf[pl.ds(c * LANES, LANES)] = acc_b[c]

            serial_pred = jnp.logical_not(jnp.logical_or(uniform, one_b))

            @pl.when(serial_pred)
            def _():
                cur0, nflush0 = state[0], state[1]
                accs0 = [accbuf[pl.ds(c * LANES, LANES)] for c in range(DG)]
                cur1, nflush1, *accs1 = lax.fori_loop(
                    0, LANES, lambda jj, ec: entry(jj, i, ec),
                    (cur0, nflush0, *accs0))
                state[0] = cur1
                state[1] = nflush1
                for c in range(DG):
                    accbuf[pl.ds(c * LANES, LANES)] = accs1[c]

            return 0

        return lax.fori_loop(0, BATCH // LANES, group, carry)

    state[0] = first_seg
    state[1] = jnp.int32(0)
    for c in range(DG):
        accbuf[pl.ds(c * LANES, LANES)] = neg
    lax.fori_loop(0, NBATCH, batch_body, 0)
    cur = state[0]
    nflush = state[1]
    accs = [accbuf[pl.ds(c * LANES, LANES)] for c in range(DG)]

    @pl.when(cur == first_seg)
    def _():
        emit_side(cur, accs, 0)

    emit_side(cur, accs, 1)

    for s in range(RING):
        @pl.when(s < nflush)
        def _():
            pltpu.make_async_copy(rowring.at[pl.ds(s * D_C, D_C)],
                                  out_hbm.at[0], fsem.at[s]).wait()


def _tc_fixup_body(ssegs_smem, draft_any, srows_vmem, ssegs_vmem, out_any,
                   merged_vmem, sem):
    del draft_any
    segs = ssegs_vmem[:, 0:1]

    def merge(j, acc):
        seg_col = ssegs_vmem[pl.ds(j, 1), 0:1]
        mask = segs == seg_col
        row_j = srows_vmem[pl.ds(j, 1), :]
        return jnp.maximum(acc, jnp.where(mask, row_j, NEG_INF))

    merged_vmem[...] = lax.fori_loop(
        0, 2 * NW, merge,
        jnp.full((2 * NW, D_C), NEG_INF, dtype=jnp.float32))

    def scatter(i, _):
        seg_i = ssegs_smem[i, 0]
        pltpu.make_async_copy(merged_vmem.at[i], out_any.at[seg_i],
                              sem.at[i]).start()
        return 0

    lax.fori_loop(0, 2 * NW, scatter, 0)

    def drain(i, _):
        seg_i = ssegs_smem[i, 0]
        pltpu.make_async_copy(merged_vmem.at[i], out_any.at[seg_i],
                              sem.at[i]).wait()
        return 0

    lax.fori_loop(0, 2 * NW, drain, 0)


@jax.jit
def _run(table, hyperedge_ids, segment_ids):
    mesh = plsc.VectorSubcoreMesh(core_axis_name="c", subcore_axis_name="s",
                                  num_cores=NC, num_subcores=NS)
    sc = pl.kernel(
        _sc_body,
        out_type=(
            jax.ShapeDtypeStruct((N_NODES_C, D_C), jnp.float32),
            jax.ShapeDtypeStruct((2 * NW, D_C), jnp.float32),
            jax.ShapeDtypeStruct((2 * NW, LANES), jnp.int32),
        ),
        mesh=mesh,
        scratch_types=[
            pltpu.VMEM((CHUNK + LANES,), jnp.int32),
            pltpu.VMEM((CHUNK,), jnp.int32),
            pltpu.VMEM((2 * BATCH, D_C), jnp.float32),
            pltpu.VMEM((RING * D_C,), jnp.float32),
            pltpu.VMEM((D_C,), jnp.float32),
            pltpu.VMEM((LANES,), jnp.int32),
            pltpu.VMEM((D_C,), jnp.float32),
            pltpu.SMEM((2,), jnp.int32),
            pltpu.SemaphoreType.DMA((2,)),
            pltpu.SemaphoreType.DMA((RING,)),
            pltpu.SemaphoreType.DMA,
        ],
    )
    draft, srows, ssegs = sc(table, hyperedge_ids, segment_ids)

    fix = pl.pallas_call(
        _tc_fixup_body,
        out_shape=jax.ShapeDtypeStruct((N_NODES_C, D_C), jnp.float32),
        in_specs=[
            pl.BlockSpec(memory_space=pltpu.SMEM),
            pl.BlockSpec(memory_space=pl.ANY),
            pl.BlockSpec(memory_space=pltpu.VMEM),
            pl.BlockSpec(memory_space=pltpu.VMEM),
        ],
        out_specs=pl.BlockSpec(memory_space=pl.ANY),
        scratch_shapes=[
            pltpu.VMEM((2 * NW, D_C), jnp.float32),
            pltpu.SemaphoreType.DMA((2 * NW,)),
        ],
        input_output_aliases={1: 0},
    )
    return fix(ssegs, draft, srows, ssegs)


def kernel(table, hyperedge_ids, segment_ids):
    return _run(table, hyperedge_ids, segment_ids)


# speedup vs baseline: 1.1811x; 1.0019x over previous
import jax
import jax.numpy as jnp
from jax import lax
from jax.experimental import pallas as pl
from jax.experimental.pallas import tpu as pltpu
from jax.experimental.pallas import tpu_sc as plsc

N_NODES_C = 10000
E_INC = 320000
D_C = 128

NC = 2
NS = 16
NW = NC * NS
LANES = 16
DG = D_C // LANES

CHUNK = E_INC // NW
BATCH = 400
SUB = 80
NSUB = BATCH // SUB
NBATCH = CHUNK // BATCH
NEG_INF = float("-inf")


RING = 8


def _sc_body(table_hbm, he_hbm, seg_hbm, out_hbm, srows_hbm, ssegs_hbm,
             seg_v, he_v, rows_v, rowring, rowbuf, segbuf, accbuf, state,
             gsem, fsem, isem):
    wid = lax.axis_index("c") * NS + lax.axis_index("s")
    start = wid * CHUNK

    cp_seg = pltpu.make_async_copy(seg_hbm.at[pl.ds(start, CHUNK)],
                                   seg_v.at[pl.ds(0, CHUNK)], isem)
    cp_he = pltpu.make_async_copy(he_hbm.at[pl.ds(start, CHUNK)], he_v, isem)
    cp_seg.start()
    cp_he.start()
    cp_seg.wait()
    cp_he.wait()

    first_seg = seg_v[pl.ds(0, LANES)][0]
    neg = jnp.full((LANES,), NEG_INF, dtype=jnp.float32)

    def emit_side(cur, accs, side_slot):
        for c in range(DG):
            rowbuf[pl.ds(c * LANES, LANES)] = accs[c]
        segbuf[...] = jnp.full((LANES,), cur, dtype=jnp.int32)
        pltpu.sync_copy(rowbuf, srows_hbm.at[2 * wid + side_slot])
        pltpu.sync_copy(segbuf, ssegs_hbm.at[2 * wid + side_slot])

    def flush_direct(cur, nflush, accs):
        slot = lax.rem(nflush, RING)

        @pl.when(nflush >= RING)
        def _():
            pltpu.make_async_copy(rowring.at[pl.ds(slot * D_C, D_C)],
                                  out_hbm.at[cur], fsem.at[slot]).wait()

        for c in range(DG):
            rowring[pl.ds(slot * D_C + c * LANES, LANES)] = accs[c]
        pltpu.make_async_copy(rowring.at[pl.ds(slot * D_C, D_C)],
                              out_hbm.at[cur], fsem.at[slot]).start()

    def fire(k, buf):
        for t in range(NSUB):
            pltpu.make_async_copy(
                table_hbm.at[he_v.at[pl.ds(k * BATCH + t * SUB, SUB)]],
                rows_v.at[pl.ds(buf * BATCH + t * SUB, SUB)],
                gsem.at[buf]).start()

    def drain(buf):
        for t in range(NSUB):
            pltpu.make_async_copy(
                table_hbm.at[he_v.at[pl.ds(t * SUB, SUB)]],
                rows_v.at[pl.ds(buf * BATCH + t * SUB, SUB)],
                gsem.at[buf]).wait()

    fire(0, 0)

    def batch_body(k, carry):
        buf = lax.rem(k, 2)
        drain(buf)

        @pl.when(k + 1 < NBATCH)
        def _():
            fire(k + 1, 1 - buf)

        def entry(jj, i0, ecarry):
            j = i0 + jj
            cur, nflush = ecarry[0], ecarry[1]
            accs = list(ecarry[2:])
            seg_j = seg_v[pl.ds(j, LANES)][0]
            new_seg = seg_j != cur
            direct = jnp.logical_and(new_seg, cur != first_seg)

            @pl.when(jnp.logical_and(new_seg, cur == first_seg))
            def _():
                emit_side(cur, accs, 0)

            @pl.when(direct)
            def _():
                flush_direct(cur, nflush, accs)

            nflush = nflush + direct.astype(jnp.int32)
            new_accs = []
            for c in range(DG):
                row_c = rows_v[buf * BATCH + j - k * BATCH,
                               pl.ds(c * LANES, LANES)]
                base = jnp.where(new_seg, neg, accs[c])
                new_accs.append(jnp.maximum(base, row_c))
            cur = jnp.where(new_seg, seg_j, cur)
            return (cur, nflush, *new_accs)

        def group(g, lo):
            i = k * BATCH + g * LANES
            sv = seg_v[pl.ds(i, LANES)]
            hi = sv[LANES - 1]
            uniform = jnp.logical_and(lo == hi, i > 0)
            svt = [sv[t] for t in range(LANES)]
            all_two = jnp.logical_or(svt[0] == lo, svt[0] == hi)
            for t in range(1, LANES):
                all_two = jnp.logical_and(
                    all_two, jnp.logical_or(svt[t] == lo, svt[t] == hi))
            one_b = jnp.logical_and(
                jnp.logical_and(jnp.logical_not(uniform), all_two), i > 0)

            @pl.when(uniform)
            def _():
                row0 = buf * BATCH + g * LANES
                for c in range(DG):
                    r = [rows_v[row0 + t, pl.ds(c * LANES, LANES)]
                         for t in range(LANES)]
                    while len(r) > 1:
                        r = [jnp.maximum(r[2 * q], r[2 * q + 1])
                             for q in range(len(r) // 2)]
                    accbuf[pl.ds(c * LANES, LANES)] = jnp.maximum(
                        accbuf[pl.ds(c * LANES, LANES)], r[0])

            @pl.when(one_b)
            def _():
                p = svt[0].astype(jnp.int32) * 0
                for t in range(LANES):
                    p = p + (svt[t] == lo).astype(jnp.int32)
                cur0, nflush0 = state[0], state[1]
                row0 = buf * BATCH + g * LANES
                tlt = [t < p for t in range(LANES)]
                acc_a = []
                acc_b = []
                for c in range(DG):
                    r = [rows_v[row0 + t, pl.ds(c * LANES, LANES)]
                         for t in range(LANES)]
                    ra = [jnp.where(tlt[t], r[t], neg) for t in range(LANES)]
                    rb = [jnp.where(tlt[t], neg, r[t]) for t in range(LANES)]
                    while len(ra) > 1:
                        ra = [jnp.maximum(ra[2 * q], ra[2 * q + 1])
                              for q in range(len(ra) // 2)]
                        rb = [jnp.maximum(rb[2 * q], rb[2 * q + 1])
                              for q in range(len(rb) // 2)]
                    acc_a.append(jnp.maximum(
                        accbuf[pl.ds(c * LANES, LANES)], ra[0]))
                    acc_b.append(rb[0])

                @pl.when(cur0 == first_seg)
                def _():
                    emit_side(cur0, acc_a, 0)

                direct = cur0 != first_seg

                @pl.when(direct)
                def _():
                    flush_direct(cur0, nflush0, acc_a)

                state[0] = hi
                state[1] = nflush0 + direct.astype(jnp.int32)
                for c in range(DG):
                    accbuf[pl.ds(c * LANES, LANES)] = acc_b[c]

            serial_pred = jnp.logical_not(jnp.logical_or(uniform, one_b))

            @pl.when(serial_pred)
            def _():
                cur0, nflush0 = state[0], state[1]
                accs0 = [accbuf[pl.ds(c * LANES, LANES)] for c in range(DG)]
                cur1, nflush1, *accs1 = lax.fori_loop(
                    0, LANES, lambda jj, ec: entry(jj, i, ec),
                    (cur0, nflush0, *accs0))
                state[0] = cur1
                state[1] = nflush1
                for c in range(DG):
                    accbuf[pl.ds(c * LANES, LANES)] = accs1[c]

            return hi

        return lax.fori_loop(0, BATCH // LANES, group, carry)

    state[0] = first_seg
    state[1] = jnp.int32(0)
    for c in range(DG):
        accbuf[pl.ds(c * LANES, LANES)] = neg
    lax.fori_loop(0, NBATCH, batch_body, first_seg)
    cur = state[0]
    nflush = state[1]
    accs = [accbuf[pl.ds(c * LANES, LANES)] for c in range(DG)]

    @pl.when(cur == first_seg)
    def _():
        emit_side(cur, accs, 0)

    emit_side(cur, accs, 1)

    for s in range(RING):
        @pl.when(s < nflush)
        def _():
            pltpu.make_async_copy(rowring.at[pl.ds(s * D_C, D_C)],
                                  out_hbm.at[0], fsem.at[s]).wait()


def _tc_fixup_body(ssegs_smem, draft_any, srows_vmem, ssegs_vmem, out_any,
                   merged_vmem, sem):
    del draft_any
    segs = ssegs_vmem[:, 0:1]

    def merge(j, acc):
        seg_col = ssegs_vmem[pl.ds(j, 1), 0:1]
        mask = segs == seg_col
        row_j = srows_vmem[pl.ds(j, 1), :]
        return jnp.maximum(acc, jnp.where(mask, row_j, NEG_INF))

    merged_vmem[...] = lax.fori_loop(
        0, 2 * NW, merge,
        jnp.full((2 * NW, D_C), NEG_INF, dtype=jnp.float32))

    def scatter(i, _):
        seg_i = ssegs_smem[i, 0]
        pltpu.make_async_copy(merged_vmem.at[i], out_any.at[seg_i],
                              sem.at[i]).start()
        return 0

    lax.fori_loop(0, 2 * NW, scatter, 0)

    def drain(i, _):
        seg_i = ssegs_smem[i, 0]
        pltpu.make_async_copy(merged_vmem.at[i], out_any.at[seg_i],
                              sem.at[i]).wait()
        return 0

    lax.fori_loop(0, 2 * NW, drain, 0)


@jax.jit
def _run(table, hyperedge_ids, segment_ids):
    mesh = plsc.VectorSubcoreMesh(core_axis_name="c", subcore_axis_name="s",
                                  num_cores=NC, num_subcores=NS)
    sc = pl.kernel(
        _sc_body,
        out_type=(
            jax.ShapeDtypeStruct((N_NODES_C, D_C), jnp.float32),
            jax.ShapeDtypeStruct((2 * NW, D_C), jnp.float32),
            jax.ShapeDtypeStruct((2 * NW, LANES), jnp.int32),
        ),
        mesh=mesh,
        scratch_types=[
            pltpu.VMEM((CHUNK + LANES,), jnp.int32),
            pltpu.VMEM((CHUNK,), jnp.int32),
            pltpu.VMEM((2 * BATCH, D_C), jnp.float32),
            pltpu.VMEM((RING * D_C,), jnp.float32),
            pltpu.VMEM((D_C,), jnp.float32),
            pltpu.VMEM((LANES,), jnp.int32),
            pltpu.VMEM((D_C,), jnp.float32),
            pltpu.SMEM((2,), jnp.int32),
            pltpu.SemaphoreType.DMA((2,)),
            pltpu.SemaphoreType.DMA((RING,)),
            pltpu.SemaphoreType.DMA,
        ],
    )
    draft, srows, ssegs = sc(table, hyperedge_ids, segment_ids)

    fix = pl.pallas_call(
        _tc_fixup_body,
        out_shape=jax.ShapeDtypeStruct((N_NODES_C, D_C), jnp.float32),
        in_specs=[
            pl.BlockSpec(memory_space=pltpu.SMEM),
            pl.BlockSpec(memory_space=pl.ANY),
            pl.BlockSpec(memory_space=pltpu.VMEM),
            pl.BlockSpec(memory_space=pltpu.VMEM),
        ],
        out_specs=pl.BlockSpec(memory_space=pl.ANY),
        scratch_shapes=[
            pltpu.VMEM((2 * NW, D_C), jnp.float32),
            pltpu.SemaphoreType.DMA((2 * NW,)),
        ],
        input_output_aliases={1: 0},
    )
    return fix(ssegs, draft, srows, ssegs)


def kernel(table, hyperedge_ids, segment_ids):
    return _run(table, hyperedge_ids, segment_ids)
